# Initial kernel scaffold; baseline (speedup 1.0000x reference)
import jax
import jax.numpy as jnp
from jax.experimental import pallas as pl


def kernel(indices, table):
    raise NotImplementedError("write your pallas kernel here")


# speedup vs baseline: 5.7746x; 5.7746x over previous
import functools

import jax
import jax.numpy as jnp
from jax import lax
from jax.experimental import pallas as pl
from jax.experimental.pallas import tpu as pltpu
from jax.experimental.pallas import tpu_sc as plsc

_VOCAB = 100000
_D = 128
_BATCH = 1024
_SEQ = 200
_B = _BATCH * _SEQ

_NC = 2
_NS = 16
_NW = _NC * _NS
_BPW = _B // _NW
_CH = 128
_NCHUNK = _BPW // _CH

_mesh = plsc.VectorSubcoreMesh(core_axis_name="c", subcore_axis_name="s")


@functools.partial(
    pl.kernel,
    mesh=_mesh,
    out_type=jax.ShapeDtypeStruct((_B, _D), jnp.float32),
    scratch_types=[
        pltpu.VMEM((_NCHUNK, _CH), jnp.int32),
        pltpu.VMEM((_CH, _D), jnp.float32),
        pltpu.SemaphoreType.DMA,
    ],
)
def _sc_gather(table_hbm, idx_hbm, out_hbm, idx_v, rows_v, gsem):
    wid = lax.axis_index("s") * _NC + lax.axis_index("c")
    base = wid * _BPW
    pltpu.sync_copy(idx_hbm.at[wid], idx_v)

    def chunk(c, carry):
        pltpu.async_copy(table_hbm.at[idx_v.at[c]], rows_v, gsem).wait()
        pltpu.sync_copy(rows_v, out_hbm.at[pl.ds(base + c * _CH, _CH)])
        return carry

    lax.fori_loop(0, _NCHUNK, chunk, 0)


def kernel(indices, table):
    idx3 = indices.reshape(_NW, _NCHUNK, _CH)
    out = _sc_gather(table, idx3)
    return out.reshape(_BATCH, _SEQ, _D)


# speedup vs baseline: 6.6203x; 1.1465x over previous
import functools

import jax
import jax.numpy as jnp
from jax import lax
from jax.experimental import pallas as pl
from jax.experimental.pallas import tpu as pltpu
from jax.experimental.pallas import tpu_sc as plsc

_VOCAB = 100000
_D = 128
_BATCH = 1024
_SEQ = 200
_B = _BATCH * _SEQ

_NC = 2
_NS = 16
_NW = _NC * _NS
_BPW = _B // _NW
_CH = 128
_NCHUNK = _BPW // _CH

_mesh = plsc.VectorSubcoreMesh(core_axis_name="c", subcore_axis_name="s")


@functools.partial(
    pl.kernel,
    mesh=_mesh,
    out_type=jax.ShapeDtypeStruct((_B, _D), jnp.float32),
    scratch_types=[
        pltpu.VMEM((_NCHUNK, _CH), jnp.int32),
        pltpu.VMEM((_CH, _D), jnp.float32),
        pltpu.VMEM((_CH, _D), jnp.float32),
        pltpu.SemaphoreType.DMA,
        pltpu.SemaphoreType.DMA,
        pltpu.SemaphoreType.DMA,
        pltpu.SemaphoreType.DMA,
    ],
)
def _sc_gather(table_hbm, idx_hbm, out_hbm, idx_v, rows0, rows1,
               gsem0, gsem1, wsem0, wsem1):
    wid = lax.axis_index("s") * _NC + lax.axis_index("c")
    base = wid * _BPW
    pltpu.sync_copy(idx_hbm.at[wid], idx_v)

    rows = (rows0, rows1)
    gsem = (gsem0, gsem1)
    wsem = (wsem0, wsem1)

    def g_copy(c, b):
        return pltpu.make_async_copy(table_hbm.at[idx_v.at[c]], rows[b], gsem[b])

    def w_copy(c, b):
        return pltpu.make_async_copy(
            rows[b], out_hbm.at[pl.ds(base + c * _CH, _CH)], wsem[b])

    g_copy(0, 0).start()
    g_copy(0, 0).wait()
    g_copy(1, 1).start()
    w_copy(0, 0).start()

    def step(o, carry):
        c1 = 2 * o + 1
        c2 = c1 + 1
        g_copy(c1, 1).wait()
        w_copy(c1 - 1, 0).wait()
        g_copy(c2, 0).start()
        w_copy(c1, 1).start()
        g_copy(c2, 0).wait()
        w_copy(c2 - 1, 1).wait()
        g_copy(c2 + 1, 1).start()
        w_copy(c2, 0).start()
        return carry

    lax.fori_loop(0, (_NCHUNK - 2) // 2, step, 0)

    g_copy(_NCHUNK - 1, 1).wait()
    w_copy(_NCHUNK - 2, 0).wait()
    w_copy(_NCHUNK - 1, 1).start()
    w_copy(_NCHUNK - 1, 1).wait()


def kernel(indices, table):
    idx3 = indices.reshape(_NW, _NCHUNK, _CH)
    out = _sc_gather(table, idx3)
    return out.reshape(_BATCH, _SEQ, _D)


# speedup vs baseline: 7.9438x; 1.1999x over previous
import functools

import jax
import jax.numpy as jnp
from jax import lax
from jax.experimental import pallas as pl
from jax.experimental.pallas import tpu as pltpu
from jax.experimental.pallas import tpu_sc as plsc

_VOCAB = 100000
_D = 128
_BATCH = 1024
_SEQ = 200
_B = _BATCH * _SEQ

_NC = 2
_NS = 16
_NW = _NC * _NS
_BPW = _B // _NW
_CH = 128
_NCHUNK = _BPW // _CH

_mesh = plsc.VectorSubcoreMesh(core_axis_name="c", subcore_axis_name="s")


@functools.partial(
    pl.kernel,
    mesh=_mesh,
    out_type=jax.ShapeDtypeStruct((_B, _D), jnp.float32),
    scratch_types=[
        pltpu.VMEM((_NCHUNK, _CH), jnp.int32),
        pltpu.VMEM((_CH, _D), jnp.float32),
        pltpu.VMEM((_CH, _D), jnp.float32),
        pltpu.VMEM((_CH, _D), jnp.float32),
        pltpu.VMEM((_CH, _D), jnp.float32),
        pltpu.SemaphoreType.DMA,
        pltpu.SemaphoreType.DMA,
        pltpu.SemaphoreType.DMA,
        pltpu.SemaphoreType.DMA,
        pltpu.SemaphoreType.DMA,
        pltpu.SemaphoreType.DMA,
        pltpu.SemaphoreType.DMA,
        pltpu.SemaphoreType.DMA,
    ],
)
def _sc_gather(table_hbm, idx_hbm, out_hbm, idx_v, rows0, rows1, rows2, rows3,
               gsem0, gsem1, gsem2, gsem3, wsem0, wsem1, wsem2, wsem3):
    wid = lax.axis_index("s") * _NC + lax.axis_index("c")
    base = wid * _BPW
    pltpu.sync_copy(idx_hbm.at[wid], idx_v)

    rows = (rows0, rows1, rows2, rows3)
    gsem = (gsem0, gsem1, gsem2, gsem3)
    wsem = (wsem0, wsem1, wsem2, wsem3)

    def g_copy(c, b):
        return pltpu.make_async_copy(table_hbm.at[idx_v.at[c]], rows[b], gsem[b])

    def w_copy(c, b):
        return pltpu.make_async_copy(
            rows[b], out_hbm.at[pl.ds(base + c * _CH, _CH)], wsem[b])

    g_copy(0, 0).start()
    g_copy(1, 1).start()
    g_copy(0, 0).wait()
    w_copy(0, 0).start()
    g_copy(2, 2).start()
    g_copy(1, 1).wait()
    w_copy(1, 1).start()
    g_copy(3, 3).start()

    def stage(c, b):
        nb = (b + 2) % 4
        g_copy(c, b).wait()
        w_copy(c, b).start()
        w_copy(c - 2, nb).wait()
        g_copy(c + 2, nb).start()

    def step(o, carry):
        c0 = 2 + 4 * o
        for cc in range(4):
            stage(c0 + cc, (2 + cc) % 4)
        return carry

    lax.fori_loop(0, (_NCHUNK - 6) // 4, step, 0)

    g_copy(46, 2).wait()
    w_copy(46, 2).start()
    w_copy(44, 0).wait()
    g_copy(48, 0).start()
    g_copy(47, 3).wait()
    w_copy(47, 3).start()
    w_copy(45, 1).wait()
    g_copy(49, 1).start()
    g_copy(48, 0).wait()
    w_copy(48, 0).start()
    g_copy(49, 1).wait()
    w_copy(49, 1).start()
    w_copy(46, 2).wait()
    w_copy(47, 3).wait()
    w_copy(48, 0).wait()
    w_copy(49, 1).wait()


def kernel(indices, table):
    idx3 = indices.reshape(_NW, _NCHUNK, _CH)
    out = _sc_gather(table, idx3)
    return out.reshape(_BATCH, _SEQ, _D)


# speedup vs baseline: 7.9625x; 1.0024x over previous
import functools

import jax
import jax.numpy as jnp
from jax import lax
from jax.experimental import pallas as pl
from jax.experimental.pallas import tpu as pltpu
from jax.experimental.pallas import tpu_sc as plsc

_VOCAB = 100000
_D = 128
_BATCH = 1024
_SEQ = 200
_B = _BATCH * _SEQ

_NC = 2
_NS = 16
_NW = _NC * _NS
_BPW = _B // _NW
_CH = 128
_NCHUNK = _BPW // _CH

_NBUF = 6
_PRE = 3

_mesh = plsc.VectorSubcoreMesh(core_axis_name="c", subcore_axis_name="s")


@functools.partial(
    pl.kernel,
    mesh=_mesh,
    out_type=jax.ShapeDtypeStruct((_B, _D), jnp.float32),
    scratch_types=(
        [pltpu.VMEM((_NCHUNK, _CH), jnp.int32)]
        + [pltpu.VMEM((_CH, _D), jnp.float32)] * _NBUF
        + [pltpu.SemaphoreType.DMA] * (2 * _NBUF)
    ),
)
def _sc_gather(table_hbm, idx_hbm, out_hbm, idx_v, *bufs):
    rows = bufs[:_NBUF]
    gsem = bufs[_NBUF:2 * _NBUF]
    wsem = bufs[2 * _NBUF:]

    wid = lax.axis_index("s") * _NC + lax.axis_index("c")
    base = wid * _BPW
    pltpu.sync_copy(idx_hbm.at[wid], idx_v)

    def g_copy(c, b):
        return pltpu.make_async_copy(table_hbm.at[idx_v.at[c]], rows[b], gsem[b])

    def w_copy(c, b):
        return pltpu.make_async_copy(
            rows[b], out_hbm.at[pl.ds(base + c * _CH, _CH)], wsem[b])

    def stage(c, b):
        nb = (b + _PRE) % _NBUF
        g_copy(c, b).wait()
        w_copy(c, b).start()
        w_copy(c - _PRE, nb).wait()
        g_copy(c + _PRE, nb).start()

    for c in range(_PRE):
        g_copy(c, c % _NBUF).start()
    for c in range(_PRE):
        b = c % _NBUF
        g_copy(c, b).wait()
        w_copy(c, b).start()
        g_copy(c + _PRE, (c + _PRE) % _NBUF).start()

    lo = _PRE
    hi = _NCHUNK - 1 - _PRE
    k = (hi - lo + 1) // _NBUF

    def step(o, carry):
        c0 = lo + o * _NBUF
        for cc in range(_NBUF):
            stage(c0 + cc, (lo + cc) % _NBUF)
        return carry

    lax.fori_loop(0, k, step, 0)

    for c in range(lo + k * _NBUF, hi + 1):
        stage(c, c % _NBUF)
    for c in range(hi + 1, _NCHUNK):
        b = c % _NBUF
        g_copy(c, b).wait()
        w_copy(c, b).start()
    for c in range(_NCHUNK - _NBUF, _NCHUNK):
        w_copy(c, c % _NBUF).wait()


def kernel(indices, table):
    idx3 = indices.reshape(_NW, _NCHUNK, _CH)
    out = _sc_gather(table, idx3)
    return out.reshape(_BATCH, _SEQ, _D)
